# Initial kernel scaffold; baseline (speedup 1.0000x reference)
import jax
import jax.numpy as jnp
from jax.experimental import pallas as pl


def kernel(pos, enc, pos_emb, numAgents_emb, num_agents, T, params):
    raise NotImplementedError("write your pallas kernel here")


# speedup vs baseline: 84.8974x; 84.8974x over previous
import functools

import jax
import jax.numpy as jnp
from jax.experimental import pallas as pl
from jax.experimental.pallas import tpu as pltpu

_B = 128
_A = 64
_EMB = 32
_POS_EMB = 16
_ENC_DIM = 128
_L = 4
_G = 4
_JB = 8


def _body(posx_ref, posy_ref, tf_ref, enc_ref, pemb_ref, na_ref,
          fc1WT_ref, fc1b_ref, fc2WT_ref, fc2b_ref,
          WleT_ref, WlpT_ref, wlna_ref, linb_ref,
          W1dT_ref, W1sT_ref, W1pT_ref, w1ts_ref, w1td_ref, b1_ref,
          W2blk_ref, b2t_ref,
          WuhT_ref, WuaT_ref, u1b_ref, Wu2T_ref, u2b_ref,
          out_ref):
    G, A, EMB, JB = _G, _A, _EMB, _JB
    NJ = A // JB

    f32 = jnp.float32
    dot = functools.partial(jnp.dot, preferred_element_type=f32)

    enc = enc_ref[...].reshape(G, _ENC_DIM)
    e1 = jnp.tanh(dot(enc, fc1WT_ref[...]) + fc1b_ref[...])
    enc_emb = dot(e1, fc2WT_ref[...]) + fc2b_ref[...]
    na = na_ref[...].reshape(G, 1)
    scene_c = dot(enc_emb, WleT_ref[...]) + na * wlna_ref[...] + linb_ref[...]

    pe = pemb_ref[...].reshape(G * A, _POS_EMB)
    h = dot(pe, WlpT_ref[...])
    h = h + jnp.broadcast_to(scene_c[:, None, :], (G, A, EMB)).reshape(G * A, EMB)

    px = posx_ref[...].reshape(G * A, 1)
    py = posy_ref[...].reshape(G * A, 1)
    tf = tf_ref[...].reshape(G * A, 1)

    iota_r = jax.lax.broadcasted_iota(jnp.int32, (G * A, JB * EMB), 0)
    iota_l = jax.lax.broadcasted_iota(jnp.int32, (G * A, JB * EMB), 1)
    pack_mask = (iota_r % JB) == (iota_l // EMB)

    def pack(x):
        tiled = jnp.concatenate([x] * JB, axis=1)
        sel = jnp.where(pack_mask, tiled, 0.0)
        return sel.reshape(G * A // JB, JB, JB * EMB).sum(axis=1)

    for l in range(_L):
        P = px * W1pT_ref[l, 0:1, :] + py * W1pT_ref[l, 1:2, :]
        D = dot(h, W1dT_ref[l]) - P + tf * w1td_ref[l] + b1_ref[l]
        S = dot(h, W1sT_ref[l]) + P + tf * w1ts_ref[l]
        S4 = pack(S).reshape(G, 1, NJ, JB * EMB)
        Dt = jnp.concatenate([D] * JB, axis=1).reshape(G, A, 1, JB * EMB)
        pre = Dt + S4
        t1 = jnp.tanh(pre).reshape(G * A * NJ, JB * EMB)
        m = jnp.tanh(dot(t1, W2blk_ref[l]) + b2t_ref[l])
        r = m.reshape(G * A, NJ, JB * EMB).sum(axis=1)
        aggr = r[:, 0:EMB]
        for k in range(1, JB):
            aggr = aggr + r[:, k * EMB:(k + 1) * EMB]
        u = jnp.tanh(dot(h, WuhT_ref[l]) + dot(aggr, WuaT_ref[l]) + u1b_ref[l])
        h = h + jnp.tanh(dot(u, Wu2T_ref[l]) + u2b_ref[l])

    out_ref[...] = h.reshape(G, A, EMB)


def kernel(pos, enc, pos_emb, numAgents_emb, num_agents, T, params):
    B, A = pos.shape[0], pos.shape[1]
    L, EMB, JB = _L, _EMB, _JB
    f32 = jnp.float32

    G = _G
    NG = B // G
    posx = pos[:, :, 0].reshape(NG, G * A, 1)
    posy = pos[:, :, 1].reshape(NG, G * A, 1)
    tf = T.astype(f32).reshape(NG, G * A, 1)
    enc3 = enc.reshape(NG, G, _ENC_DIM)
    na3 = numAgents_emb.reshape(NG, G, 1)

    fc1W, fc1b = params["fc1"]
    fc2W, fc2b = params["fc2"]
    linW, linb = params["lin_in"]
    lay = params["layers"]
    msg1W = jnp.stack([lay[l]["msg1"][0] for l in range(L)])
    msg1b = jnp.stack([lay[l]["msg1"][1] for l in range(L)])
    msg2W = jnp.stack([lay[l]["msg2"][0] for l in range(L)])
    msg2b = jnp.stack([lay[l]["msg2"][1] for l in range(L)])
    upd1W = jnp.stack([lay[l]["upd1"][0] for l in range(L)])
    upd1b = jnp.stack([lay[l]["upd1"][1] for l in range(L)])
    upd2W = jnp.stack([lay[l]["upd2"][0] for l in range(L)])
    upd2b = jnp.stack([lay[l]["upd2"][1] for l in range(L)])

    tr = lambda w: jnp.transpose(w, (0, 2, 1))
    W1dT = tr(msg1W[:, :, 0:EMB])
    W1sT = tr(msg1W[:, :, EMB:2 * EMB])
    W1pT = tr(msg1W[:, :, 2 * EMB:2 * EMB + 2])
    w1ts = msg1W[:, None, :, 2 * EMB + 2]
    w1td = msg1W[:, None, :, 2 * EMB + 3]
    b1 = msg1b[:, None, :]

    W2T = tr(msg2W)
    eye = jnp.eye(JB, dtype=f32)
    W2blk = jax.vmap(lambda w: jnp.kron(eye, w))(W2T)
    b2t = jnp.tile(msg2b, (1, JB))[:, None, :]

    WuhT = tr(upd1W[:, :, 0:EMB])
    WuaT = tr(upd1W[:, :, EMB:2 * EMB])
    u1b = upd1b[:, None, :]
    Wu2T = tr(upd2W)
    u2b = upd2b[:, None, :]

    WleT = linW[:, 0:EMB].T
    WlpT = linW[:, EMB:EMB + _POS_EMB].T
    wlna = linW[None, :, EMB + _POS_EMB]
    linb2 = linb[None, :]

    grid = (NG,)

    def bs(block, imap):
        return pl.BlockSpec(block, imap)

    row3 = lambda i: (i, 0, 0)
    full2 = lambda i: (0, 0)
    full3 = lambda i: (0, 0, 0)

    in_specs = [
        bs((1, G * A, 1), row3),
        bs((1, G * A, 1), row3),
        bs((1, G * A, 1), row3),
        bs((1, G, _ENC_DIM), row3),
        bs((G, A, _POS_EMB), row3),
        bs((1, G, 1), row3),
        bs(fc1W.T.shape, full2), bs((1, fc1b.shape[0]), full2),
        bs(fc2W.T.shape, full2), bs((1, fc2b.shape[0]), full2),
        bs((EMB, EMB), full2), bs((_POS_EMB, EMB), full2),
        bs((1, EMB), full2), bs((1, EMB), full2),
        bs((L, EMB, EMB), full3), bs((L, EMB, EMB), full3),
        bs((L, 2, EMB), full3), bs((L, 1, EMB), full3),
        bs((L, 1, EMB), full3), bs((L, 1, EMB), full3),
        bs((L, JB * EMB, JB * EMB), full3), bs((L, 1, JB * EMB), full3),
        bs((L, EMB, EMB), full3), bs((L, EMB, EMB), full3),
        bs((L, 1, EMB), full3),
        bs((L, EMB, EMB), full3), bs((L, 1, EMB), full3),
    ]

    out = pl.pallas_call(
        _body,
        grid=grid,
        in_specs=in_specs,
        out_specs=pl.BlockSpec((G, A, EMB), row3),
        out_shape=jax.ShapeDtypeStruct((B, A, EMB), f32),
        compiler_params=pltpu.CompilerParams(
            dimension_semantics=("parallel",),
        ),
    )(posx, posy, tf, enc3, pos_emb, na3,
      fc1W.T, fc1b[None, :], fc2W.T, fc2b[None, :],
      WleT, WlpT, wlna, linb2,
      W1dT, W1sT, W1pT, w1ts, w1td, b1,
      W2blk, b2t,
      WuhT, WuaT, u1b, Wu2T, u2b)
    return out


# speedup vs baseline: 139.1304x; 1.6388x over previous
import functools

import jax
import jax.numpy as jnp
from jax.experimental import pallas as pl
from jax.experimental.pallas import tpu as pltpu

_B = 128
_A = 64
_EMB = 32
_POS_EMB = 16
_ENC_DIM = 128
_L = 4
_G = 4
_JB = 8


def _body(posx_ref, posy_ref, tf_ref, enc_ref, pemb_ref, na_ref,
          fc1WT_ref, fc1b_ref, fc2WT_ref, fc2b_ref,
          WleT_ref, WlpT_ref, wlna_ref, linb_ref,
          W1dT_ref, W1sT_ref, W1pT_ref, w1ts_ref, w1td_ref, b1_ref,
          W2blk_ref, b2t_ref,
          WuhT_ref, WuaT_ref, u1b_ref, Wu2T_ref, u2b_ref,
          Esel_ref, TileEye_ref, Q_ref, F_ref,
          out_ref):
    G, A, EMB, JB = _G, _A, _EMB, _JB
    NJ = A // JB
    W = JB * EMB

    f32 = jnp.float32
    dot = functools.partial(jnp.dot, preferred_element_type=f32)

    enc = enc_ref[...].reshape(G, _ENC_DIM)
    e1 = jnp.tanh(dot(enc, fc1WT_ref[...]) + fc1b_ref[...])
    enc_emb = dot(e1, fc2WT_ref[...]) + fc2b_ref[...]
    na = na_ref[...].reshape(G, 1)
    scene_c = dot(enc_emb, WleT_ref[...]) + na * wlna_ref[...] + linb_ref[...]

    pe = pemb_ref[...].reshape(G * A, _POS_EMB)
    h = dot(pe, WlpT_ref[...]) + dot(Esel_ref[...], scene_c)

    px = posx_ref[...].reshape(G * A, 1)
    py = posy_ref[...].reshape(G * A, 1)
    tf = tf_ref[...].reshape(G * A, 1)

    iota_r = jax.lax.broadcasted_iota(jnp.int32, (G * A, W), 0)
    iota_l = jax.lax.broadcasted_iota(jnp.int32, (G * A, W), 1)
    pack_mask = (iota_r % JB) == (iota_l // EMB)
    zeros_w = jnp.zeros((G * A, W), f32)

    for l in range(_L):
        P = px * W1pT_ref[l, 0:1, :] + py * W1pT_ref[l, 1:2, :]
        D = dot(h, W1dT_ref[l]) - P + tf * w1td_ref[l] + b1_ref[l]
        S = dot(h, W1sT_ref[l]) + P + tf * w1ts_ref[l]
        S_masked = jnp.where(pack_mask, jnp.concatenate([S] * JB, axis=1),
                             zeros_w)
        Sb = dot(Q_ref[...], S_masked)
        Dt = dot(D, TileEye_ref[...])
        Db = jnp.broadcast_to(Dt.reshape(G, 1, A, W), (G, NJ, A, W))
        pre = Db + Sb.reshape(G, NJ, A, W)
        t1 = jnp.tanh(pre).reshape(G * A * NJ, W)
        m = jnp.tanh(dot(t1, W2blk_ref[l]) + b2t_ref[l])
        r = m.reshape(G, NJ, A, W).sum(axis=1).reshape(G * A, W)
        aggr = dot(r, F_ref[...])
        u = jnp.tanh(dot(h, WuhT_ref[l]) + dot(aggr, WuaT_ref[l]) + u1b_ref[l])
        h = h + jnp.tanh(dot(u, Wu2T_ref[l]) + u2b_ref[l])

    out_ref[...] = h.reshape(G, A, EMB)


def kernel(pos, enc, pos_emb, numAgents_emb, num_agents, T, params):
    B, A = pos.shape[0], pos.shape[1]
    L, EMB, JB = _L, _EMB, _JB
    NJ = A // JB
    f32 = jnp.float32

    G = _G
    NG = B // G
    posx = pos[:, :, 0].reshape(NG, G * A, 1)
    posy = pos[:, :, 1].reshape(NG, G * A, 1)
    tf = T.astype(f32).reshape(NG, G * A, 1)
    enc3 = enc.reshape(NG, G, _ENC_DIM)
    na3 = numAgents_emb.reshape(NG, G, 1)

    fc1W, fc1b = params["fc1"]
    fc2W, fc2b = params["fc2"]
    linW, linb = params["lin_in"]
    lay = params["layers"]
    msg1W = jnp.stack([lay[l]["msg1"][0] for l in range(L)])
    msg1b = jnp.stack([lay[l]["msg1"][1] for l in range(L)])
    msg2W = jnp.stack([lay[l]["msg2"][0] for l in range(L)])
    msg2b = jnp.stack([lay[l]["msg2"][1] for l in range(L)])
    upd1W = jnp.stack([lay[l]["upd1"][0] for l in range(L)])
    upd1b = jnp.stack([lay[l]["upd1"][1] for l in range(L)])
    upd2W = jnp.stack([lay[l]["upd2"][0] for l in range(L)])
    upd2b = jnp.stack([lay[l]["upd2"][1] for l in range(L)])

    tr = lambda w: jnp.transpose(w, (0, 2, 1))
    W1dT = tr(msg1W[:, :, 0:EMB])
    W1sT = tr(msg1W[:, :, EMB:2 * EMB])
    W1pT = tr(msg1W[:, :, 2 * EMB:2 * EMB + 2])
    w1ts = msg1W[:, None, :, 2 * EMB + 2]
    w1td = msg1W[:, None, :, 2 * EMB + 3]
    b1 = msg1b[:, None, :]

    W2T = tr(msg2W)
    eyeJ = jnp.eye(JB, dtype=f32)
    W2blk = jax.vmap(lambda w: jnp.kron(eyeJ, w))(W2T)
    b2t = jnp.tile(msg2b, (1, JB))[:, None, :]

    WuhT = tr(upd1W[:, :, 0:EMB])
    WuaT = tr(upd1W[:, :, EMB:2 * EMB])
    u1b = upd1b[:, None, :]
    Wu2T = tr(upd2W)
    u2b = upd2b[:, None, :]

    WleT = linW[:, 0:EMB].T
    WlpT = linW[:, EMB:EMB + _POS_EMB].T
    wlna = linW[None, :, EMB + _POS_EMB]
    linb2 = linb[None, :]

    eye32 = jnp.eye(EMB, dtype=f32)
    Esel = jnp.kron(jnp.eye(G, dtype=f32), jnp.ones((A, 1), f32))
    TileEye = jnp.kron(jnp.ones((1, JB), f32), eye32)
    Q = jnp.kron(jnp.eye(G * NJ, dtype=f32), jnp.ones((A, JB), f32))
    F = jnp.kron(jnp.ones((JB, 1), f32), eye32)

    grid = (NG,)
    WW = JB * EMB

    def bs(block, imap):
        return pl.BlockSpec(block, imap)

    row3 = lambda i: (i, 0, 0)
    full2 = lambda i: (0, 0)
    full3 = lambda i: (0, 0, 0)

    in_specs = [
        bs((1, G * A, 1), row3),
        bs((1, G * A, 1), row3),
        bs((1, G * A, 1), row3),
        bs((1, G, _ENC_DIM), row3),
        bs((G, A, _POS_EMB), row3),
        bs((1, G, 1), row3),
        bs(fc1W.T.shape, full2), bs((1, fc1b.shape[0]), full2),
        bs(fc2W.T.shape, full2), bs((1, fc2b.shape[0]), full2),
        bs((EMB, EMB), full2), bs((_POS_EMB, EMB), full2),
        bs((1, EMB), full2), bs((1, EMB), full2),
        bs((L, EMB, EMB), full3), bs((L, EMB, EMB), full3),
        bs((L, 2, EMB), full3), bs((L, 1, EMB), full3),
        bs((L, 1, EMB), full3), bs((L, 1, EMB), full3),
        bs((L, WW, WW), full3), bs((L, 1, WW), full3),
        bs((L, EMB, EMB), full3), bs((L, EMB, EMB), full3),
        bs((L, 1, EMB), full3),
        bs((L, EMB, EMB), full3), bs((L, 1, EMB), full3),
        bs(Esel.shape, full2), bs(TileEye.shape, full2),
        bs(Q.shape, full2), bs(F.shape, full2),
    ]

    out = pl.pallas_call(
        _body,
        grid=grid,
        in_specs=in_specs,
        out_specs=pl.BlockSpec((G, A, EMB), row3),
        out_shape=jax.ShapeDtypeStruct((B, A, EMB), f32),
        compiler_params=pltpu.CompilerParams(
            dimension_semantics=("parallel",),
        ),
    )(posx, posy, tf, enc3, pos_emb, na3,
      fc1W.T, fc1b[None, :], fc2W.T, fc2b[None, :],
      WleT, WlpT, wlna, linb2,
      W1dT, W1sT, W1pT, w1ts, w1td, b1,
      W2blk, b2t,
      WuhT, WuaT, u1b, Wu2T, u2b,
      Esel, TileEye, Q, F)
    return out


# speedup vs baseline: 194.5815x; 1.3986x over previous
import functools

import jax
import jax.numpy as jnp
from jax.experimental import pallas as pl
from jax.experimental.pallas import tpu as pltpu

_B = 128
_A = 64
_EMB = 32
_POS_EMB = 16
_ENC_DIM = 128
_L = 4
_G = 16
_JB = 8


def _body(posx_ref, posy_ref, tf_ref, enc_ref, pemb_ref, na_ref,
          fc1WT_ref, fc1b_ref, fc2WT_ref, fc2b_ref,
          WleT_ref, WlpT_ref, wlna_ref, linb_ref,
          W1dT_ref, W1sT_ref, W1pT_ref, w1ts_ref, w1td_ref, b1_ref,
          W2blk_ref, b2t_ref,
          WuhT_ref, WuaT_ref, u1b_ref, Wu2T_ref, u2b_ref,
          Esel_ref, TileEye_ref, Q_ref, F_ref,
          out_ref):
    G, A, EMB, JB = _G, _A, _EMB, _JB
    NJ = A // JB
    W = JB * EMB

    f32 = jnp.float32
    dot = functools.partial(jnp.dot, preferred_element_type=f32)

    enc = enc_ref[...].reshape(G, _ENC_DIM)
    e1 = jnp.tanh(dot(enc, fc1WT_ref[...]) + fc1b_ref[...])
    enc_emb = dot(e1, fc2WT_ref[...]) + fc2b_ref[...]
    na = na_ref[...].reshape(G, 1)
    scene_c = dot(enc_emb, WleT_ref[...]) + na * wlna_ref[...] + linb_ref[...]

    pe = pemb_ref[...].reshape(G * A, _POS_EMB)
    h = dot(pe, WlpT_ref[...]) + dot(Esel_ref[...], scene_c)

    px = posx_ref[...].reshape(G * A, 1)
    py = posy_ref[...].reshape(G * A, 1)
    tf = tf_ref[...].reshape(G * A, 1)

    iota_r = jax.lax.broadcasted_iota(jnp.int32, (G * A, W), 0)
    iota_l = jax.lax.broadcasted_iota(jnp.int32, (G * A, W), 1)
    pack_mask = (iota_r % JB) == (iota_l // EMB)
    zeros_w = jnp.zeros((G * A, W), f32)

    for l in range(_L):
        P = px * W1pT_ref[l, 0:1, :] + py * W1pT_ref[l, 1:2, :]
        D = dot(h, W1dT_ref[l]) - P + tf * w1td_ref[l] + b1_ref[l]
        S = dot(h, W1sT_ref[l]) + P + tf * w1ts_ref[l]
        S_masked = jnp.where(pack_mask, jnp.concatenate([S] * JB, axis=1),
                             zeros_w)
        S4 = dot(Q_ref[...], S_masked)
        Dt = dot(D, TileEye_ref[...])
        Db = jnp.broadcast_to(Dt.reshape(G, 1, A, W), (G, NJ, A, W))
        pre = Db + S4.reshape(G, NJ, 1, W)
        t1 = jnp.tanh(pre).reshape(G * A * NJ, W)
        m = jnp.tanh(dot(t1, W2blk_ref[l]) + b2t_ref[l])
        r = m.reshape(G, NJ, A, W).sum(axis=1).reshape(G * A, W)
        aggr = dot(r, F_ref[...])
        u = jnp.tanh(dot(h, WuhT_ref[l]) + dot(aggr, WuaT_ref[l]) + u1b_ref[l])
        h = h + jnp.tanh(dot(u, Wu2T_ref[l]) + u2b_ref[l])

    out_ref[...] = h.reshape(G, A, EMB)


def kernel(pos, enc, pos_emb, numAgents_emb, num_agents, T, params):
    B, A = pos.shape[0], pos.shape[1]
    L, EMB, JB = _L, _EMB, _JB
    NJ = A // JB
    f32 = jnp.float32

    G = _G
    NG = B // G
    posx = pos[:, :, 0].reshape(NG, G * A, 1)
    posy = pos[:, :, 1].reshape(NG, G * A, 1)
    tf = T.astype(f32).reshape(NG, G * A, 1)
    enc3 = enc.reshape(NG, G, _ENC_DIM)
    na3 = numAgents_emb.reshape(NG, G, 1)

    fc1W, fc1b = params["fc1"]
    fc2W, fc2b = params["fc2"]
    linW, linb = params["lin_in"]
    lay = params["layers"]
    msg1W = jnp.stack([lay[l]["msg1"][0] for l in range(L)])
    msg1b = jnp.stack([lay[l]["msg1"][1] for l in range(L)])
    msg2W = jnp.stack([lay[l]["msg2"][0] for l in range(L)])
    msg2b = jnp.stack([lay[l]["msg2"][1] for l in range(L)])
    upd1W = jnp.stack([lay[l]["upd1"][0] for l in range(L)])
    upd1b = jnp.stack([lay[l]["upd1"][1] for l in range(L)])
    upd2W = jnp.stack([lay[l]["upd2"][0] for l in range(L)])
    upd2b = jnp.stack([lay[l]["upd2"][1] for l in range(L)])

    tr = lambda w: jnp.transpose(w, (0, 2, 1))
    W1dT = tr(msg1W[:, :, 0:EMB])
    W1sT = tr(msg1W[:, :, EMB:2 * EMB])
    W1pT = tr(msg1W[:, :, 2 * EMB:2 * EMB + 2])
    w1ts = msg1W[:, None, :, 2 * EMB + 2]
    w1td = msg1W[:, None, :, 2 * EMB + 3]
    b1 = msg1b[:, None, :]

    W2T = tr(msg2W)
    eyeJ = jnp.eye(JB, dtype=f32)
    W2blk = jax.vmap(lambda w: jnp.kron(eyeJ, w))(W2T)
    b2t = jnp.tile(msg2b, (1, JB))[:, None, :]

    WuhT = tr(upd1W[:, :, 0:EMB])
    WuaT = tr(upd1W[:, :, EMB:2 * EMB])
    u1b = upd1b[:, None, :]
    Wu2T = tr(upd2W)
    u2b = upd2b[:, None, :]

    WleT = linW[:, 0:EMB].T
    WlpT = linW[:, EMB:EMB + _POS_EMB].T
    wlna = linW[None, :, EMB + _POS_EMB]
    linb2 = linb[None, :]

    eye32 = jnp.eye(EMB, dtype=f32)
    Esel = jnp.kron(jnp.eye(G, dtype=f32), jnp.ones((A, 1), f32))
    TileEye = jnp.kron(jnp.ones((1, JB), f32), eye32)
    Q = jnp.kron(jnp.eye(G * NJ, dtype=f32), jnp.ones((1, JB), f32))
    F = jnp.kron(jnp.ones((JB, 1), f32), eye32)

    grid = (NG,)
    WW = JB * EMB

    def bs(block, imap):
        return pl.BlockSpec(block, imap)

    row3 = lambda i: (i, 0, 0)
    full2 = lambda i: (0, 0)
    full3 = lambda i: (0, 0, 0)

    in_specs = [
        bs((1, G * A, 1), row3),
        bs((1, G * A, 1), row3),
        bs((1, G * A, 1), row3),
        bs((1, G, _ENC_DIM), row3),
        bs((G, A, _POS_EMB), row3),
        bs((1, G, 1), row3),
        bs(fc1W.T.shape, full2), bs((1, fc1b.shape[0]), full2),
        bs(fc2W.T.shape, full2), bs((1, fc2b.shape[0]), full2),
        bs((EMB, EMB), full2), bs((_POS_EMB, EMB), full2),
        bs((1, EMB), full2), bs((1, EMB), full2),
        bs((L, EMB, EMB), full3), bs((L, EMB, EMB), full3),
        bs((L, 2, EMB), full3), bs((L, 1, EMB), full3),
        bs((L, 1, EMB), full3), bs((L, 1, EMB), full3),
        bs((L, WW, WW), full3), bs((L, 1, WW), full3),
        bs((L, EMB, EMB), full3), bs((L, EMB, EMB), full3),
        bs((L, 1, EMB), full3),
        bs((L, EMB, EMB), full3), bs((L, 1, EMB), full3),
        bs(Esel.shape, full2), bs(TileEye.shape, full2),
        bs(Q.shape, full2), bs(F.shape, full2),
    ]

    out = pl.pallas_call(
        _body,
        grid=grid,
        in_specs=in_specs,
        out_specs=pl.BlockSpec((G, A, EMB), row3),
        out_shape=jax.ShapeDtypeStruct((B, A, EMB), f32),
        compiler_params=pltpu.CompilerParams(
            dimension_semantics=("parallel",),
        ),
    )(posx, posy, tf, enc3, pos_emb, na3,
      fc1W.T, fc1b[None, :], fc2W.T, fc2b[None, :],
      WleT, WlpT, wlna, linb2,
      W1dT, W1sT, W1pT, w1ts, w1td, b1,
      W2blk, b2t,
      WuhT, WuaT, u1b, Wu2T, u2b,
      Esel, TileEye, Q, F)
    return out


# speedup vs baseline: 200.1120x; 1.0284x over previous
import functools

import jax
import jax.numpy as jnp
import numpy as np
from jax.experimental import pallas as pl
from jax.experimental.pallas import tpu as pltpu

_B = 128
_A = 64
_EMB = 32
_POS_EMB = 16
_ENC_DIM = 128
_L = 4
_G = 16
_JB = 8


def _body(posx_ref, posy_ref, tf_ref, enc_ref, pemb_ref, na_ref,
          fc1WT_ref, fc1b_ref, fc2WT_ref, fc2b_ref,
          WleT_ref, WlpT_ref, wlna_ref, linb_ref,
          W1dT_ref, W1sT_ref, W1pT_ref, w1ts_ref, w1td_ref, b1_ref,
          W2blk_ref, b2t_ref,
          WuhT_ref, WuaT_ref, u1b_ref, Wu2T_ref, u2b_ref,
          Esel_ref, TileEye_ref, Q_ref, F_ref,
          out_ref):
    G, A, EMB, JB = _G, _A, _EMB, _JB
    NJ = A // JB
    W = JB * EMB

    f32 = jnp.float32
    dot = functools.partial(jnp.dot, preferred_element_type=f32)

    enc = enc_ref[...].reshape(G, _ENC_DIM)
    e1 = jnp.tanh(dot(enc, fc1WT_ref[...]) + fc1b_ref[...])
    enc_emb = dot(e1, fc2WT_ref[...]) + fc2b_ref[...]
    na = na_ref[...].reshape(G, 1)
    scene_c = dot(enc_emb, WleT_ref[...]) + na * wlna_ref[...] + linb_ref[...]

    pe = pemb_ref[...].reshape(G * A, _POS_EMB)
    h = dot(pe, WlpT_ref[...]) + dot(Esel_ref[...], scene_c)

    px = posx_ref[...].reshape(G * A, 1)
    py = posy_ref[...].reshape(G * A, 1)
    tf = tf_ref[...].reshape(G * A, 1)

    iota_r = jax.lax.broadcasted_iota(jnp.int32, (G * A, W), 0)
    iota_l = jax.lax.broadcasted_iota(jnp.int32, (G * A, W), 1)
    pack_mask = (iota_r % JB) == (iota_l // EMB)
    zeros_w = jnp.zeros((G * A, W), f32)

    for l in range(_L):
        P = px * W1pT_ref[l, 0:1, :] + py * W1pT_ref[l, 1:2, :]
        D = dot(h, W1dT_ref[l]) - P + tf * w1td_ref[l] + b1_ref[l]
        S = dot(h, W1sT_ref[l]) + P + tf * w1ts_ref[l]
        S_masked = jnp.where(pack_mask, jnp.concatenate([S] * JB, axis=1),
                             zeros_w)
        S4 = dot(Q_ref[...], S_masked)
        Dt = dot(D, TileEye_ref[...])
        Db = jnp.broadcast_to(Dt.reshape(G, 1, A, W), (G, NJ, A, W))
        pre = Db + S4.reshape(G, NJ, 1, W)
        t1 = jnp.tanh(pre).reshape(G * A * NJ, W)
        m = jnp.tanh(dot(t1, W2blk_ref[l]) + b2t_ref[l])
        r = m.reshape(G, NJ, A, W).sum(axis=1).reshape(G * A, W)
        aggr = dot(r, F_ref[...])
        u = jnp.tanh(dot(h, WuhT_ref[l]) + dot(aggr, WuaT_ref[l]) + u1b_ref[l])
        h = h + jnp.tanh(dot(u, Wu2T_ref[l]) + u2b_ref[l])

    out_ref[...] = h.reshape(G, A, EMB)


def kernel(pos, enc, pos_emb, numAgents_emb, num_agents, T, params):
    B, A = pos.shape[0], pos.shape[1]
    L, EMB, JB = _L, _EMB, _JB
    NJ = A // JB
    f32 = jnp.float32

    G = _G
    NG = B // G
    posx = pos[:, :, 0].reshape(NG, G * A, 1)
    posy = pos[:, :, 1].reshape(NG, G * A, 1)
    tf = T.astype(f32).reshape(NG, G * A, 1)
    enc3 = enc.reshape(NG, G, _ENC_DIM)
    na3 = numAgents_emb.reshape(NG, G, 1)

    fc1W, fc1b = params["fc1"]
    fc2W, fc2b = params["fc2"]
    linW, linb = params["lin_in"]
    lay = params["layers"]
    msg1W = jnp.stack([lay[l]["msg1"][0] for l in range(L)])
    msg1b = jnp.stack([lay[l]["msg1"][1] for l in range(L)])
    msg2W = jnp.stack([lay[l]["msg2"][0] for l in range(L)])
    msg2b = jnp.stack([lay[l]["msg2"][1] for l in range(L)])
    upd1W = jnp.stack([lay[l]["upd1"][0] for l in range(L)])
    upd1b = jnp.stack([lay[l]["upd1"][1] for l in range(L)])
    upd2W = jnp.stack([lay[l]["upd2"][0] for l in range(L)])
    upd2b = jnp.stack([lay[l]["upd2"][1] for l in range(L)])

    msg1WT = jnp.transpose(msg1W, (0, 2, 1))
    W1dT = msg1WT[:, 0:EMB, :]
    W1sT = msg1WT[:, EMB:2 * EMB, :]
    W1pT = msg1WT[:, 2 * EMB:2 * EMB + 2, :]
    w1ts = msg1WT[:, None, 2 * EMB + 2, :]
    w1td = msg1WT[:, None, 2 * EMB + 3, :]
    b1 = msg1b[:, None, :]

    W2T = jnp.transpose(msg2W, (0, 2, 1))
    eyeJ = jnp.asarray(np.eye(JB, dtype=np.float32))
    W2blk = jax.vmap(lambda w: jnp.kron(eyeJ, w))(W2T)
    b2t = jnp.tile(msg2b, (1, JB))[:, None, :]

    upd1WT = jnp.transpose(upd1W, (0, 2, 1))
    WuhT = upd1WT[:, 0:EMB, :]
    WuaT = upd1WT[:, EMB:2 * EMB, :]
    u1b = upd1b[:, None, :]
    Wu2T = jnp.transpose(upd2W, (0, 2, 1))
    u2b = upd2b[:, None, :]

    linWT = linW.T
    WleT = linWT[0:EMB, :]
    WlpT = linWT[EMB:EMB + _POS_EMB, :]
    wlna = linWT[None, EMB + _POS_EMB, :]
    linb2 = linb[None, :]

    eye32 = np.eye(EMB, dtype=np.float32)
    Esel = jnp.asarray(np.kron(np.eye(G, dtype=np.float32),
                               np.ones((A, 1), np.float32)))
    TileEye = jnp.asarray(np.kron(np.ones((1, JB), np.float32), eye32))
    Q = jnp.asarray(np.kron(np.eye(G * NJ, dtype=np.float32),
                            np.ones((1, JB), np.float32)))
    F = jnp.asarray(np.kron(np.ones((JB, 1), np.float32), eye32))

    grid = (NG,)
    WW = JB * EMB

    def bs(block, imap):
        return pl.BlockSpec(block, imap)

    row3 = lambda i: (i, 0, 0)
    full2 = lambda i: (0, 0)
    full3 = lambda i: (0, 0, 0)

    in_specs = [
        bs((1, G * A, 1), row3),
        bs((1, G * A, 1), row3),
        bs((1, G * A, 1), row3),
        bs((1, G, _ENC_DIM), row3),
        bs((G, A, _POS_EMB), row3),
        bs((1, G, 1), row3),
        bs(fc1W.T.shape, full2), bs((1, fc1b.shape[0]), full2),
        bs(fc2W.T.shape, full2), bs((1, fc2b.shape[0]), full2),
        bs((EMB, EMB), full2), bs((_POS_EMB, EMB), full2),
        bs((1, EMB), full2), bs((1, EMB), full2),
        bs((L, EMB, EMB), full3), bs((L, EMB, EMB), full3),
        bs((L, 2, EMB), full3), bs((L, 1, EMB), full3),
        bs((L, 1, EMB), full3), bs((L, 1, EMB), full3),
        bs((L, WW, WW), full3), bs((L, 1, WW), full3),
        bs((L, EMB, EMB), full3), bs((L, EMB, EMB), full3),
        bs((L, 1, EMB), full3),
        bs((L, EMB, EMB), full3), bs((L, 1, EMB), full3),
        bs(Esel.shape, full2), bs(TileEye.shape, full2),
        bs(Q.shape, full2), bs(F.shape, full2),
    ]

    out = pl.pallas_call(
        _body,
        grid=grid,
        in_specs=in_specs,
        out_specs=pl.BlockSpec((G, A, EMB), row3),
        out_shape=jax.ShapeDtypeStruct((B, A, EMB), f32),
        compiler_params=pltpu.CompilerParams(
            dimension_semantics=("parallel",),
        ),
    )(posx, posy, tf, enc3, pos_emb, na3,
      fc1W.T, fc1b[None, :], fc2W.T, fc2b[None, :],
      WleT, WlpT, wlna, linb2,
      W1dT, W1sT, W1pT, w1ts, w1td, b1,
      W2blk, b2t,
      WuhT, WuaT, u1b, Wu2T, u2b,
      Esel, TileEye, Q, F)
    return out


# speedup vs baseline: 224.1908x; 1.1203x over previous
import functools

import jax
import jax.numpy as jnp
import numpy as np
from jax.experimental import pallas as pl
from jax.experimental.pallas import tpu as pltpu

_B = 128
_A = 64
_EMB = 32
_POS_EMB = 16
_ENC_DIM = 128
_L = 4
_G = 16
_JB = 8


def _body(posx_ref, posy_ref, tf_ref, enc_ref, pemb_ref, na_ref,
          fc1WT_ref, fc1b_ref, fc2WT_ref, fc2b_ref,
          WleT_ref, WlpT_ref, wlna_ref, linb_ref,
          W1dsT_ref, W1pT_ref, w1ts_ref, w1td_ref, b1_ref,
          W2blk_ref, b2t_ref,
          WuT_ref, u1b_ref, Wu2T_ref, u2b_ref,
          Esel_ref, TileEye_ref, Q_ref, F_ref,
          out_ref):
    G, A, EMB, JB = _G, _A, _EMB, _JB
    NJ = A // JB
    W = JB * EMB

    f32 = jnp.float32
    dot = functools.partial(jnp.dot, preferred_element_type=f32)

    enc = enc_ref[...].reshape(G, _ENC_DIM)
    e1 = jnp.tanh(dot(enc, fc1WT_ref[...]) + fc1b_ref[...])
    enc_emb = dot(e1, fc2WT_ref[...]) + fc2b_ref[...]
    na = na_ref[...].reshape(G, 1)
    scene_c = dot(enc_emb, WleT_ref[...]) + na * wlna_ref[...] + linb_ref[...]

    pe = pemb_ref[...].reshape(G * A, _POS_EMB)
    h = dot(pe, WlpT_ref[...]) + dot(Esel_ref[...], scene_c)

    px = posx_ref[...].reshape(G * A, 1)
    py = posy_ref[...].reshape(G * A, 1)
    tf = tf_ref[...].reshape(G * A, 1)

    iota_r = jax.lax.broadcasted_iota(jnp.int32, (G * A, W), 0)
    iota_l = jax.lax.broadcasted_iota(jnp.int32, (G * A, W), 1)
    pack_mask = (iota_r % JB) == (iota_l // EMB)
    zeros_w = jnp.zeros((G * A, W), f32)

    for l in range(_L):
        P = px * W1pT_ref[l, 0:1, :] + py * W1pT_ref[l, 1:2, :]
        DS = dot(h, W1dsT_ref[l])
        D = DS[:, 0:EMB] - P + tf * w1td_ref[l] + b1_ref[l]
        S = DS[:, EMB:2 * EMB] + P + tf * w1ts_ref[l]
        S_masked = jnp.where(pack_mask, jnp.concatenate([S] * JB, axis=1),
                             zeros_w)
        S4 = dot(Q_ref[...], S_masked).reshape(G, NJ, W)
        Dt = dot(D, TileEye_ref[...]).reshape(G, A, W)
        acc = zeros_w
        for jj in range(NJ):
            pre = Dt + S4[:, jj:jj + 1, :]
            t1 = jnp.tanh(pre).reshape(G * A, W)
            acc = acc + jnp.tanh(dot(t1, W2blk_ref[l]) + b2t_ref[l])
        aggr = dot(acc, F_ref[...])
        uin = jnp.concatenate([h, aggr], axis=1)
        u = jnp.tanh(dot(uin, WuT_ref[l]) + u1b_ref[l])
        h = h + jnp.tanh(dot(u, Wu2T_ref[l]) + u2b_ref[l])

    out_ref[...] = h.reshape(G, A, EMB)


def kernel(pos, enc, pos_emb, numAgents_emb, num_agents, T, params):
    B, A = pos.shape[0], pos.shape[1]
    L, EMB, JB = _L, _EMB, _JB
    NJ = A // JB
    f32 = jnp.float32

    G = _G
    NG = B // G
    posx = pos[:, :, 0].reshape(NG, G * A, 1)
    posy = pos[:, :, 1].reshape(NG, G * A, 1)
    tf = T.astype(f32).reshape(NG, G * A, 1)
    enc3 = enc.reshape(NG, G, _ENC_DIM)
    na3 = numAgents_emb.reshape(NG, G, 1)

    fc1W, fc1b = params["fc1"]
    fc2W, fc2b = params["fc2"]
    linW, linb = params["lin_in"]
    lay = params["layers"]
    msg1W = jnp.stack([lay[l]["msg1"][0] for l in range(L)])
    msg1b = jnp.stack([lay[l]["msg1"][1] for l in range(L)])
    msg2W = jnp.stack([lay[l]["msg2"][0] for l in range(L)])
    msg2b = jnp.stack([lay[l]["msg2"][1] for l in range(L)])
    upd1W = jnp.stack([lay[l]["upd1"][0] for l in range(L)])
    upd1b = jnp.stack([lay[l]["upd1"][1] for l in range(L)])
    upd2W = jnp.stack([lay[l]["upd2"][0] for l in range(L)])
    upd2b = jnp.stack([lay[l]["upd2"][1] for l in range(L)])

    msg1WT = jnp.transpose(msg1W, (0, 2, 1))
    W1dsT = jnp.concatenate(
        [msg1WT[:, 0:EMB, :], msg1WT[:, EMB:2 * EMB, :]], axis=2)
    W1pT = msg1WT[:, 2 * EMB:2 * EMB + 2, :]
    w1ts = msg1WT[:, None, 2 * EMB + 2, :]
    w1td = msg1WT[:, None, 2 * EMB + 3, :]
    b1 = msg1b[:, None, :]

    W2T = jnp.transpose(msg2W, (0, 2, 1))
    eyeJ = jnp.asarray(np.eye(JB, dtype=np.float32))
    W2blk = jax.vmap(lambda w: jnp.kron(eyeJ, w))(W2T)
    b2t = jnp.tile(msg2b, (1, JB))[:, None, :]

    WuT = jnp.transpose(upd1W, (0, 2, 1))
    u1b = upd1b[:, None, :]
    Wu2T = jnp.transpose(upd2W, (0, 2, 1))
    u2b = upd2b[:, None, :]

    linWT = linW.T
    WleT = linWT[0:EMB, :]
    WlpT = linWT[EMB:EMB + _POS_EMB, :]
    wlna = linWT[None, EMB + _POS_EMB, :]
    linb2 = linb[None, :]

    eye32 = np.eye(EMB, dtype=np.float32)
    Esel = jnp.asarray(np.kron(np.eye(G, dtype=np.float32),
                               np.ones((A, 1), np.float32)))
    TileEye = jnp.asarray(np.kron(np.ones((1, JB), np.float32), eye32))
    Q = jnp.asarray(np.kron(np.eye(G * NJ, dtype=np.float32),
                            np.ones((1, JB), np.float32)))
    F = jnp.asarray(np.kron(np.ones((JB, 1), np.float32), eye32))

    grid = (NG,)
    WW = JB * EMB

    def bs(block, imap):
        return pl.BlockSpec(block, imap)

    row3 = lambda i: (i, 0, 0)
    full2 = lambda i: (0, 0)
    full3 = lambda i: (0, 0, 0)

    in_specs = [
        bs((1, G * A, 1), row3),
        bs((1, G * A, 1), row3),
        bs((1, G * A, 1), row3),
        bs((1, G, _ENC_DIM), row3),
        bs((G, A, _POS_EMB), row3),
        bs((1, G, 1), row3),
        bs(fc1W.T.shape, full2), bs((1, fc1b.shape[0]), full2),
        bs(fc2W.T.shape, full2), bs((1, fc2b.shape[0]), full2),
        bs((EMB, EMB), full2), bs((_POS_EMB, EMB), full2),
        bs((1, EMB), full2), bs((1, EMB), full2),
        bs((L, EMB, 2 * EMB), full3),
        bs((L, 2, EMB), full3), bs((L, 1, EMB), full3),
        bs((L, 1, EMB), full3), bs((L, 1, EMB), full3),
        bs((L, WW, WW), full3), bs((L, 1, WW), full3),
        bs((L, 2 * EMB, EMB), full3), bs((L, 1, EMB), full3),
        bs((L, EMB, EMB), full3), bs((L, 1, EMB), full3),
        bs(Esel.shape, full2), bs(TileEye.shape, full2),
        bs(Q.shape, full2), bs(F.shape, full2),
    ]

    out = pl.pallas_call(
        _body,
        grid=grid,
        in_specs=in_specs,
        out_specs=pl.BlockSpec((G, A, EMB), row3),
        out_shape=jax.ShapeDtypeStruct((B, A, EMB), f32),
        compiler_params=pltpu.CompilerParams(
            dimension_semantics=("parallel",),
        ),
    )(posx, posy, tf, enc3, pos_emb, na3,
      fc1W.T, fc1b[None, :], fc2W.T, fc2b[None, :],
      WleT, WlpT, wlna, linb2,
      W1dsT, W1pT, w1ts, w1td, b1,
      W2blk, b2t,
      WuT, u1b, Wu2T, u2b,
      Esel, TileEye, Q, F)
    return out


# speedup vs baseline: 243.1997x; 1.0848x over previous
import functools

import jax
import jax.numpy as jnp
import numpy as np
from jax.experimental import pallas as pl
from jax.experimental.pallas import tpu as pltpu

_B = 128
_A = 64
_EMB = 32
_POS_EMB = 16
_ENC_DIM = 128
_L = 4
_G = 16
_JB = 8


def _dotT(x, w):
    return jax.lax.dot_general(x, w, (((1,), (1,)), ((), ())),
                               preferred_element_type=jnp.float32)


def _body(pos_ref, tf_ref, enc_ref, pemb_ref, na_ref,
          fc1aug_ref, fc2aug_ref, linaug_ref, Wlp_ref,
          Waug_ref, W2blk_ref, b2t_ref, Wuaug_ref, Wu2aug_ref,
          Esel_ref, TileEye_ref, Q_ref, F_ref,
          out_ref):
    G, A, EMB, JB = _G, _A, _EMB, _JB
    NJ = A // JB
    W = JB * EMB

    f32 = jnp.float32
    dot = functools.partial(jnp.dot, preferred_element_type=f32)

    enc = enc_ref[...]
    na = na_ref[...]
    onesG = jnp.ones((G, 1), f32)
    e1 = jnp.tanh(_dotT(jnp.concatenate([enc, onesG], axis=1), fc1aug_ref[...]))
    enc_emb = _dotT(jnp.concatenate([e1, onesG], axis=1), fc2aug_ref[...])
    scene_c = _dotT(jnp.concatenate([enc_emb, na, onesG], axis=1),
                    linaug_ref[...])

    pe = pemb_ref[...].reshape(G * A, _POS_EMB)
    h = _dotT(pe, Wlp_ref[...]) + dot(Esel_ref[...], scene_c)

    pos2 = pos_ref[...].reshape(G * A, 2)
    tf = tf_ref[...].reshape(G * A, 1)
    ones1 = jnp.ones((G * A, 1), f32)

    iota_r = jax.lax.broadcasted_iota(jnp.int32, (G * A, W), 0)
    iota_l = jax.lax.broadcasted_iota(jnp.int32, (G * A, W), 1)
    pack_mask = (iota_r % JB) == (iota_l // EMB)
    zeros_w = jnp.zeros((G * A, W), f32)

    for l in range(_L):
        hx = jnp.concatenate([h, pos2, tf, ones1], axis=1)
        DS = _dotT(hx, Waug_ref[l])
        D = DS[:, 0:EMB]
        S = DS[:, EMB:2 * EMB]
        S_masked = jnp.where(pack_mask, jnp.concatenate([S] * JB, axis=1),
                             zeros_w)
        S4 = dot(Q_ref[...], S_masked).reshape(G, NJ, W)
        Dt = dot(D, TileEye_ref[...]).reshape(G, A, W)
        acc = zeros_w
        for jj in range(NJ):
            pre = Dt + S4[:, jj:jj + 1, :]
            t1 = jnp.tanh(pre).reshape(G * A, W)
            acc = acc + jnp.tanh(_dotT(t1, W2blk_ref[l]) + b2t_ref[l])
        aggr = dot(acc, F_ref[...])
        u = jnp.tanh(_dotT(jnp.concatenate([h, aggr, ones1], axis=1),
                           Wuaug_ref[l]))
        h = h + jnp.tanh(_dotT(jnp.concatenate([u, ones1], axis=1),
                               Wu2aug_ref[l]))

    out_ref[...] = h.reshape(G, A, EMB)


def kernel(pos, enc, pos_emb, numAgents_emb, num_agents, T, params):
    B, A = pos.shape[0], pos.shape[1]
    L, EMB, JB = _L, _EMB, _JB
    NJ = A // JB
    f32 = jnp.float32

    G = _G
    NG = B // G
    tf = T.astype(f32).reshape(NG, G * A, 1)

    fc1W, fc1b = params["fc1"]
    fc2W, fc2b = params["fc2"]
    linW, linb = params["lin_in"]
    lay = params["layers"]
    msg1W = jnp.stack([lay[l]["msg1"][0] for l in range(L)])
    msg1b = jnp.stack([lay[l]["msg1"][1] for l in range(L)])
    msg2W = jnp.stack([lay[l]["msg2"][0] for l in range(L)])
    msg2b = jnp.stack([lay[l]["msg2"][1] for l in range(L)])
    upd1W = jnp.stack([lay[l]["upd1"][0] for l in range(L)])
    upd1b = jnp.stack([lay[l]["upd1"][1] for l in range(L)])
    upd2W = jnp.stack([lay[l]["upd2"][0] for l in range(L)])
    upd2b = jnp.stack([lay[l]["upd2"][1] for l in range(L)])

    zL = jnp.zeros((L, EMB, 1), f32)
    Wd_aug = jnp.concatenate(
        [msg1W[:, :, 0:EMB], -msg1W[:, :, 2 * EMB:2 * EMB + 2],
         msg1W[:, :, 2 * EMB + 3:2 * EMB + 4], msg1b[:, :, None]], axis=2)
    Ws_aug = jnp.concatenate(
        [msg1W[:, :, EMB:2 * EMB], msg1W[:, :, 2 * EMB:2 * EMB + 2],
         msg1W[:, :, 2 * EMB + 2:2 * EMB + 3], zL], axis=2)
    Waug = jnp.concatenate([Wd_aug, Ws_aug], axis=1)

    blockmask = jnp.asarray(np.kron(np.eye(JB, dtype=np.float32),
                                    np.ones((EMB, EMB), np.float32)))
    ww = jnp.concatenate([msg2W] * JB, axis=2)
    W2blk = jnp.concatenate([ww] * JB, axis=1) * blockmask
    b2t = jnp.tile(msg2b, (1, JB))[:, None, :]

    Wuaug = jnp.concatenate([upd1W, upd1b[:, :, None]], axis=2)
    Wu2aug = jnp.concatenate([upd2W, upd2b[:, :, None]], axis=2)

    fc1aug = jnp.concatenate([fc1W, fc1b[:, None]], axis=1)
    fc2aug = jnp.concatenate([fc2W, fc2b[:, None]], axis=1)
    linaug = jnp.concatenate(
        [linW[:, 0:EMB], linW[:, EMB + _POS_EMB:EMB + _POS_EMB + 1],
         linb[:, None]], axis=1)
    Wlp = linW[:, EMB:EMB + _POS_EMB]

    eye32 = np.eye(EMB, dtype=np.float32)
    Esel = jnp.asarray(np.kron(np.eye(G, dtype=np.float32),
                               np.ones((A, 1), np.float32)))
    TileEye = jnp.asarray(np.kron(np.ones((1, JB), np.float32), eye32))
    Q = jnp.asarray(np.kron(np.eye(G * NJ, dtype=np.float32),
                            np.ones((1, JB), np.float32)))
    F = jnp.asarray(np.kron(np.ones((JB, 1), np.float32), eye32))

    grid = (NG,)
    WW = JB * EMB

    def bs(block, imap):
        return pl.BlockSpec(block, imap)

    row2 = lambda i: (i, 0)
    row3 = lambda i: (i, 0, 0)
    full2 = lambda i: (0, 0)
    full3 = lambda i: (0, 0, 0)

    in_specs = [
        bs((G, A, 2), row3),
        bs((1, G * A, 1), row3),
        bs((G, _ENC_DIM), row2),
        bs((G, A, _POS_EMB), row3),
        bs((G, 1), row2),
        bs(fc1aug.shape, full2), bs(fc2aug.shape, full2),
        bs(linaug.shape, full2), bs(Wlp.shape, full2),
        bs((L, 2 * EMB, EMB + 4), full3),
        bs((L, WW, WW), full3), bs((L, 1, WW), full3),
        bs((L, EMB, 2 * EMB + 1), full3), bs((L, EMB, EMB + 1), full3),
        bs(Esel.shape, full2), bs(TileEye.shape, full2),
        bs(Q.shape, full2), bs(F.shape, full2),
    ]

    out = pl.pallas_call(
        _body,
        grid=grid,
        in_specs=in_specs,
        out_specs=pl.BlockSpec((G, A, EMB), row3),
        out_shape=jax.ShapeDtypeStruct((B, A, EMB), f32),
        compiler_params=pltpu.CompilerParams(
            dimension_semantics=("parallel",),
        ),
    )(pos, tf, enc, pos_emb, numAgents_emb,
      fc1aug, fc2aug, linaug, Wlp,
      Waug, W2blk, b2t, Wuaug, Wu2aug,
      Esel, TileEye, Q, F)
    return out


# speedup vs baseline: 282.6458x; 1.1622x over previous
import functools

import jax
import jax.numpy as jnp
import numpy as np
from jax.experimental import pallas as pl
from jax.experimental.pallas import tpu as pltpu

_B = 128
_A = 64
_EMB = 32
_POS_EMB = 16
_ENC_DIM = 128
_L = 4
_G = 16
_JB = 8


def _dotT(x, w):
    return jax.lax.dot_general(x, w, (((1,), (1,)), ((), ())),
                               preferred_element_type=jnp.float32)


def _body(*refs):
    (pos_ref, tf_ref, enc_ref, pemb_ref, na_ref,
     fc1W_ref, fc1b_ref, fc2W_ref, fc2b_ref, linW_ref, linb_ref) = refs[:11]
    layer_refs = refs[11:11 + 8 * _L]
    Esel_ref, TileEye_ref, Q_ref, F_ref, out_ref = refs[11 + 8 * _L:]

    G, A, EMB, JB = _G, _A, _EMB, _JB
    NJ = A // JB
    W = JB * EMB

    f32 = jnp.float32
    dot = functools.partial(jnp.dot, preferred_element_type=f32)

    enc = enc_ref[...]
    na = na_ref[...]
    e1 = jnp.tanh(_dotT(enc, fc1W_ref[...]) + fc1b_ref[...])
    enc_emb = _dotT(e1, fc2W_ref[...]) + fc2b_ref[...]
    linW = linW_ref[...]
    lin_en = jnp.concatenate(
        [linW[:, 0:EMB], linW[:, EMB + _POS_EMB:EMB + _POS_EMB + 1]], axis=1)
    scene_c = _dotT(jnp.concatenate([enc_emb, na], axis=1), lin_en) \
        + linb_ref[...]

    pe = pemb_ref[...].reshape(G * A, _POS_EMB)
    h = _dotT(pe, linW[:, EMB:EMB + _POS_EMB]) + dot(Esel_ref[...], scene_c)

    pos2 = pos_ref[...].reshape(G * A, 2)
    tf = tf_ref[...].reshape(G * A, 1)

    iota_r = jax.lax.broadcasted_iota(jnp.int32, (G * A, W), 0)
    iota_l = jax.lax.broadcasted_iota(jnp.int32, (G * A, W), 1)
    pack_mask = (iota_r % JB) == (iota_l // EMB)
    blk_r = jax.lax.broadcasted_iota(jnp.int32, (W, W), 0)
    blk_l = jax.lax.broadcasted_iota(jnp.int32, (W, W), 1)
    blk_mask = (blk_r // EMB) == (blk_l // EMB)
    zeros_w = jnp.zeros((G * A, W), f32)
    zeros_blk = jnp.zeros((W, W), f32)

    hx = None
    for l in range(_L):
        (m1W_ref, m1b_ref, m2W_ref, m2b_ref,
         u1W_ref, u1b_ref, u2W_ref, u2b_ref) = layer_refs[8 * l:8 * l + 8]
        m1W = m1W_ref[...]
        Wd = jnp.concatenate(
            [m1W[:, 0:EMB], -m1W[:, 2 * EMB:2 * EMB + 2],
             m1W[:, 2 * EMB + 3:2 * EMB + 4]], axis=1)
        Ws = jnp.concatenate(
            [m1W[:, EMB:2 * EMB], m1W[:, 2 * EMB:2 * EMB + 3]], axis=1)
        Wds = jnp.concatenate([Wd, Ws], axis=0)
        hx = jnp.concatenate([h, pos2, tf], axis=1)
        DS = _dotT(hx, Wds)
        D = DS[:, 0:EMB] + m1b_ref[...]
        S = DS[:, EMB:2 * EMB]
        cc = jnp.concatenate([m2W_ref[...]] * JB, axis=1)
        W2blk = jnp.where(blk_mask, jnp.concatenate([cc] * JB, axis=0),
                          zeros_blk)
        b2t = jnp.concatenate([m2b_ref[...]] * JB, axis=1)
        S_masked = jnp.where(pack_mask, jnp.concatenate([S] * JB, axis=1),
                             zeros_w)
        S4 = dot(Q_ref[...], S_masked).reshape(G, NJ, W)
        Dt = dot(D, TileEye_ref[...]).reshape(G, A, W)
        acc = zeros_w
        for jj in range(NJ):
            pre = Dt + S4[:, jj:jj + 1, :]
            t1 = jnp.tanh(pre).reshape(G * A, W)
            acc = acc + jnp.tanh(_dotT(t1, W2blk) + b2t)
        aggr = dot(acc, F_ref[...])
        uin = jnp.concatenate([h, aggr], axis=1)
        u = jnp.tanh(_dotT(uin, u1W_ref[...]) + u1b_ref[...])
        h = h + jnp.tanh(_dotT(u, u2W_ref[...]) + u2b_ref[...])

    out_ref[...] = h.reshape(G, A, EMB)


def kernel(pos, enc, pos_emb, numAgents_emb, num_agents, T, params):
    B, A = pos.shape[0], pos.shape[1]
    L, EMB, JB = _L, _EMB, _JB
    NJ = A // JB
    f32 = jnp.float32

    G = _G
    NG = B // G
    tf = T.astype(f32).reshape(NG, G * A, 1)

    fc1W, fc1b = params["fc1"]
    fc2W, fc2b = params["fc2"]
    linW, linb = params["lin_in"]
    lay = params["layers"]

    layer_ops = []
    layer_specs = []

    def bs(block, imap):
        return pl.BlockSpec(block, imap)

    full2 = lambda i: (0, 0)

    for l in range(L):
        for name in ("msg1", "msg2", "upd1", "upd2"):
            Wl, bl = lay[l][name]
            layer_ops += [Wl, bl[None, :]]
            layer_specs += [bs(Wl.shape, full2), bs((1, bl.shape[0]), full2)]

    eye32 = np.eye(EMB, dtype=np.float32)
    Esel = jnp.asarray(np.kron(np.eye(G, dtype=np.float32),
                               np.ones((A, 1), np.float32)))
    TileEye = jnp.asarray(np.kron(np.ones((1, JB), np.float32), eye32))
    Q = jnp.asarray(np.kron(np.eye(G * NJ, dtype=np.float32),
                            np.ones((1, JB), np.float32)))
    F = jnp.asarray(np.kron(np.ones((JB, 1), np.float32), eye32))

    grid = (NG,)

    row2 = lambda i: (i, 0)
    row3 = lambda i: (i, 0, 0)

    in_specs = [
        bs((G, A, 2), row3),
        bs((1, G * A, 1), row3),
        bs((G, _ENC_DIM), row2),
        bs((G, A, _POS_EMB), row3),
        bs((G, 1), row2),
        bs(fc1W.shape, full2), bs((1, fc1b.shape[0]), full2),
        bs(fc2W.shape, full2), bs((1, fc2b.shape[0]), full2),
        bs(linW.shape, full2), bs((1, linb.shape[0]), full2),
    ] + layer_specs + [
        bs(Esel.shape, full2), bs(TileEye.shape, full2),
        bs(Q.shape, full2), bs(F.shape, full2),
    ]

    out = pl.pallas_call(
        _body,
        grid=grid,
        in_specs=in_specs,
        out_specs=pl.BlockSpec((G, A, EMB), row3),
        out_shape=jax.ShapeDtypeStruct((B, A, EMB), f32),
        compiler_params=pltpu.CompilerParams(
            dimension_semantics=("parallel",),
        ),
    )(pos, tf, enc, pos_emb, numAgents_emb,
      fc1W, fc1b[None, :], fc2W, fc2b[None, :], linW, linb[None, :],
      *layer_ops,
      Esel, TileEye, Q, F)
    return out
